# baseline (speedup 1.0000x reference)
import functools

import jax
import jax.numpy as jnp
from jax import lax
from jax.experimental import pallas as pl
from jax.experimental.pallas import tpu as pltpu
from jax.experimental.pallas import tpu_sc as plsc

H = 384
W = 384
C = 192
CW = C // 2
N = H * W
NC = 2
NS = 16
NW = NC * NS
LANES = 16
N_PER_W = N // NW
P = 64
NCHUNK = N_PER_W // P


def _scratch_types():
    per_set = (
        [pltpu.VMEM((P,), jnp.int32)] * 4
        + [pltpu.VMEM((P,), jnp.float32)] * 4
        + [pltpu.VMEM((P, CW), jnp.int32)] * 4
        + [pltpu.VMEM((C, P), jnp.float32)]
    )
    return ([pltpu.VMEM((N_PER_W,), jnp.float32)] * 2
            + per_set * 2
            + [pltpu.SemaphoreType.DMA] * 8
            + [pltpu.SemaphoreType.DMA] * 2)


def _make_sc_kernel():
    mesh = plsc.VectorSubcoreMesh(core_axis_name="c", subcore_axis_name="s")

    @functools.partial(
        pl.kernel,
        mesh=mesh,
        compiler_params=pltpu.CompilerParams(use_tc_tiling_on_sc=False, needs_layout_passes=False),
        out_type=jax.ShapeDtypeStruct((NW * NCHUNK, C, P), jnp.float32),
        scratch_types=_scratch_types(),
    )
    def bilinear_sc(table_hbm, xs_hbm, ys_hbm, out_hbm, *s):
        it = iter(s)
        xall = next(it)
        yall = next(it)
        idx, wts, rows, ov = [], [], [], []
        for _ in range(2):
            idx.append([next(it) for _ in range(4)])
            wts.append([next(it) for _ in range(4)])
            rows.append([next(it) for _ in range(4)])
            ov.append(next(it))
        sg = [[next(it) for _ in range(4)] for _ in range(2)]
        so = [next(it) for _ in range(2)]

        wid = lax.axis_index("s") * NC + lax.axis_index("c")
        base = wid * N_PER_W

        def fire(k, b):
            for i in range(P // LANES):
                sl = pl.ds(i * LANES, LANES)
                cs = pl.ds(k * P + i * LANES, LANES)
                x = xall[cs]
                y = yall[cs]
                x0i = jnp.minimum(jnp.maximum(x.astype(jnp.int32), 0), H - 1)
                y0i = jnp.minimum(jnp.maximum(y.astype(jnp.int32), 0), W - 1)
                x1i = jnp.minimum(x0i + 1, H - 1)
                y1i = jnp.minimum(y0i + 1, W - 1)
                xc = jnp.minimum(jnp.maximum(x, 0.0), float(H - 1))
                yc = jnp.minimum(jnp.maximum(y, 0.0), float(W - 1))
                x0f = x0i.astype(jnp.float32)
                x1f = x1i.astype(jnp.float32)
                y0f = y0i.astype(jnp.float32)
                y1f = y1i.astype(jnp.float32)
                idx[b][0][sl] = x0i * W + y0i
                idx[b][1][sl] = x0i * W + y1i
                idx[b][2][sl] = x1i * W + y0i
                idx[b][3][sl] = x1i * W + y1i
                wts[b][0][sl] = (x1f - xc) * (y1f - yc)
                wts[b][1][sl] = (x1f - xc) * (yc - y0f)
                wts[b][2][sl] = (xc - x0f) * (y1f - yc)
                wts[b][3][sl] = (xc - x0f) * (yc - y0f)
            for c in range(0):
                pltpu.async_copy(table_hbm.at[idx[b][c]], rows[b][c], sg[b][c])

        def wait_gathers(b):
            for c in range(0):
                pltpu.make_async_copy(
                    table_hbm.at[idx[b][c]], rows[b][c], sg[b][c]).wait()

        def combine(b):
            ar, br, cr, dr = rows[b]

            for g in range(P // LANES):
                gs = pl.ds(g * LANES, LANES)
                wa = wts[b][0][gs]
                wb = wts[b][1][gs]
                wc = wts[b][2][gs]
                wd = wts[b][3][gs]
                pvec = g * LANES + lax.iota(jnp.int32, LANES)

                @plsc.parallel_loop(0, CW, unroll=8)
                def _cp_body(cp):
                    cvec = jnp.full((LANES,), cp, jnp.int32)
                    a2 = plsc.unpack(plsc.bitcast(
                        plsc.load_gather(ar, [pvec, cvec]), jnp.bfloat16),
                        format=plsc.PackFormat.INTERLEAVED)
                    b2 = plsc.unpack(plsc.bitcast(
                        plsc.load_gather(br, [pvec, cvec]), jnp.bfloat16),
                        format=plsc.PackFormat.INTERLEAVED)
                    c2_ = plsc.unpack(plsc.bitcast(
                        plsc.load_gather(cr, [pvec, cvec]), jnp.bfloat16),
                        format=plsc.PackFormat.INTERLEAVED)
                    d2 = plsc.unpack(plsc.bitcast(
                        plsc.load_gather(dr, [pvec, cvec]), jnp.bfloat16),
                        format=plsc.PackFormat.INTERLEAVED)
                    for h in range(2):
                        ov[b][2 * cp + h, gs] = (
                            wa * a2[h] + wb * b2[h] + wc * c2_[h] + wd * d2[h])

        def fire_out(k, b):
            pltpu.async_copy(ov[b], out_hbm.at[wid * NCHUNK + k], so[b])

        def wait_out(b):
            pltpu.make_async_copy(
                ov[b], out_hbm.at[wid * NCHUNK], so[b]).wait()

        pltpu.sync_copy(xs_hbm.at[pl.ds(base, N_PER_W)], xall)
        pltpu.sync_copy(ys_hbm.at[pl.ds(base, N_PER_W)], yall)
        fire(0, 0)

        def pair_body(kk, carry):
            for bph in range(2):
                k = 2 * kk + bph

                @pl.when(k + 1 < NCHUNK)
                def _fire_next():
                    fire(k + 1, 1 - bph)

                wait_gathers(bph)

                @pl.when(k >= 2)
                def _drain_out():
                    wait_out(bph)

                combine(bph)
                fire_out(k, bph)
            return carry

        lax.fori_loop(0, NCHUNK // 2, pair_body, 0)
        wait_out(0)
        wait_out(1)

    return bilinear_sc


_bilinear_sc = _make_sc_kernel()


def kernel(images, coordinates):
    B, c, h, w = images.shape
    table_bf = images.reshape(c, h * w).T.astype(jnp.bfloat16)
    table_words = jax.lax.bitcast_convert_type(
        table_bf.reshape(h * w, c // 2, 2), jnp.int32)
    xs = coordinates[:, 0].reshape(-1)
    ys = coordinates[:, 1].reshape(-1)
    out_blocks = _bilinear_sc(table_words, xs, ys)
    out_cn = out_blocks.transpose(1, 0, 2).reshape(c, h * w)
    new_h, new_w = coordinates.shape[2], coordinates.shape[3]
    return out_cn.reshape(B, c, new_h, new_w)


# speedup vs baseline: 1.4862x; 1.4862x over previous
import functools

import jax
import jax.numpy as jnp
from jax import lax
from jax.experimental import pallas as pl
from jax.experimental.pallas import tpu as pltpu
from jax.experimental.pallas import tpu_sc as plsc

H = 384
W = 384
C = 192
CW = C // 2
N = H * W
NC = 2
NS = 16
NW = NC * NS
LANES = 16
N_PER_W = N // NW
P = 64
NCHUNK = N_PER_W // P


def _scratch_types():
    per_set = (
        [pltpu.VMEM((P,), jnp.int32)] * 4
        + [pltpu.VMEM((P,), jnp.float32)] * 4
        + [pltpu.VMEM((P, CW), jnp.int32)] * 4
        + [pltpu.VMEM((P, CW), jnp.int32)]
    )
    return ([pltpu.VMEM((N_PER_W,), jnp.float32)] * 2
            + per_set * 2
            + [pltpu.SemaphoreType.DMA] * 8
            + [pltpu.SemaphoreType.DMA] * 2)


def _make_sc_kernel():
    mesh = plsc.VectorSubcoreMesh(core_axis_name="c", subcore_axis_name="s")

    @functools.partial(
        pl.kernel,
        mesh=mesh,
        compiler_params=pltpu.CompilerParams(
            use_tc_tiling_on_sc=False, needs_layout_passes=False),
        out_type=jax.ShapeDtypeStruct((N, CW), jnp.int32),
        scratch_types=_scratch_types(),
    )
    def bilinear_sc(table_hbm, xs_hbm, ys_hbm, out_hbm, *s):
        it = iter(s)
        xall = next(it)
        yall = next(it)
        idx, wts, rows, ov = [], [], [], []
        for _ in range(2):
            idx.append([next(it) for _ in range(4)])
            wts.append([next(it) for _ in range(4)])
            rows.append([next(it) for _ in range(4)])
            ov.append(next(it))
        sg = [[next(it) for _ in range(4)] for _ in range(2)]
        so = [next(it) for _ in range(2)]

        wid = lax.axis_index("s") * NC + lax.axis_index("c")
        base = wid * N_PER_W

        def fire(k, b):
            for i in range(P // LANES):
                sl = pl.ds(i * LANES, LANES)
                cs = pl.ds(k * P + i * LANES, LANES)
                x = xall[cs]
                y = yall[cs]
                x0i = jnp.minimum(jnp.maximum(x.astype(jnp.int32), 0), H - 1)
                y0i = jnp.minimum(jnp.maximum(y.astype(jnp.int32), 0), W - 1)
                x1i = jnp.minimum(x0i + 1, H - 1)
                y1i = jnp.minimum(y0i + 1, W - 1)
                xc = jnp.minimum(jnp.maximum(x, 0.0), float(H - 1))
                yc = jnp.minimum(jnp.maximum(y, 0.0), float(W - 1))
                x0f = x0i.astype(jnp.float32)
                x1f = x1i.astype(jnp.float32)
                y0f = y0i.astype(jnp.float32)
                y1f = y1i.astype(jnp.float32)
                idx[b][0][sl] = x0i * W + y0i
                idx[b][1][sl] = x0i * W + y1i
                idx[b][2][sl] = x1i * W + y0i
                idx[b][3][sl] = x1i * W + y1i
                wts[b][0][sl] = (x1f - xc) * (y1f - yc)
                wts[b][1][sl] = (x1f - xc) * (yc - y0f)
                wts[b][2][sl] = (xc - x0f) * (y1f - yc)
                wts[b][3][sl] = (xc - x0f) * (yc - y0f)
            for c in range(4):
                pltpu.async_copy(table_hbm.at[idx[b][c]], rows[b][c], sg[b][c])

        def wait_gathers(b):
            for c in range(4):
                pltpu.make_async_copy(
                    table_hbm.at[idx[b][c]], rows[b][c], sg[b][c]).wait()

        def combine(b):
            ar, br, cr, dr = rows[b]

            def grp_body(g, c2):
                gs = pl.ds(g * LANES, LANES)
                wa16 = wts[b][0][gs]
                wb16 = wts[b][1][gs]
                wc16 = wts[b][2][gs]
                wd16 = wts[b][3][gs]
                row0 = g * LANES
                for p in range(LANES):
                    wav = jnp.full((LANES,), wa16[p], jnp.float32)
                    wbv = jnp.full((LANES,), wb16[p], jnp.float32)
                    wcv = jnp.full((LANES,), wc16[p], jnp.float32)
                    wdv = jnp.full((LANES,), wd16[p], jnp.float32)
                    wabf = plsc.pack(wav, wav,
                                     format=plsc.PackFormat.INTERLEAVED)
                    wbbf = plsc.pack(wbv, wbv,
                                     format=plsc.PackFormat.INTERLEAVED)
                    wcbf = plsc.pack(wcv, wcv,
                                     format=plsc.PackFormat.INTERLEAVED)
                    wdbf = plsc.pack(wdv, wdv,
                                     format=plsc.PackFormat.INTERLEAVED)
                    r = row0 + p
                    for j in range(CW // LANES):
                        sj = pl.ds(j * LANES, LANES)
                        va = plsc.bitcast(ar[r, sj], jnp.bfloat16)
                        vb = plsc.bitcast(br[r, sj], jnp.bfloat16)
                        vc = plsc.bitcast(cr[r, sj], jnp.bfloat16)
                        vd = plsc.bitcast(dr[r, sj], jnp.bfloat16)
                        acc = (wabf * va + wbbf * vb
                               + wcbf * vc + wdbf * vd)
                        ov[b][r, sj] = plsc.bitcast(acc, jnp.int32)
                return c2

            lax.fori_loop(0, P // LANES, grp_body, 0)

        def fire_out(k, b):
            off = base + k * P
            pltpu.async_copy(ov[b], out_hbm.at[pl.ds(off, P)], so[b])

        def wait_out(b):
            pltpu.make_async_copy(
                ov[b], out_hbm.at[pl.ds(base, P)], so[b]).wait()

        pltpu.sync_copy(xs_hbm.at[pl.ds(base, N_PER_W)], xall)
        pltpu.sync_copy(ys_hbm.at[pl.ds(base, N_PER_W)], yall)
        fire(0, 0)

        def pair_body(kk, carry):
            for bph in range(2):
                k = 2 * kk + bph

                @pl.when(k + 1 < NCHUNK)
                def _fire_next():
                    fire(k + 1, 1 - bph)

                wait_gathers(bph)

                @pl.when(k >= 2)
                def _drain_out():
                    wait_out(bph)

                combine(bph)
                fire_out(k, bph)
            return carry

        lax.fori_loop(0, NCHUNK // 2, pair_body, 0)
        wait_out(0)
        wait_out(1)

    return bilinear_sc


_bilinear_sc = _make_sc_kernel()


def kernel(images, coordinates):
    B, c, h, w = images.shape
    table_bf = images.reshape(c, h * w).T.astype(jnp.bfloat16)
    table_words = jax.lax.bitcast_convert_type(
        table_bf.reshape(h * w, c // 2, 2), jnp.int32)
    xs = coordinates[:, 0].reshape(-1)
    ys = coordinates[:, 1].reshape(-1)
    out_words = _bilinear_sc(table_words, xs, ys)
    out_bf = jax.lax.bitcast_convert_type(
        out_words, jnp.bfloat16).reshape(h * w, c)
    out_nc = out_bf.astype(jnp.float32)
    new_h, new_w = coordinates.shape[2], coordinates.shape[3]
    return out_nc.T.reshape(B, c, new_h, new_w)


# speedup vs baseline: 1.8363x; 1.2356x over previous
import functools

import jax
import jax.numpy as jnp
from jax import lax
from jax.experimental import pallas as pl
from jax.experimental.pallas import tpu as pltpu
from jax.experimental.pallas import tpu_sc as plsc

H = 384
W = 384
C = 192
CW = C // 2
N = H * W
NC = 2
NS = 16
NW = NC * NS
LANES = 16
N_PER_W = N // NW
P = 64
NCHUNK = N_PER_W // P


def _scratch_types():
    per_set = (
        [pltpu.VMEM((P,), jnp.int32)] * 4
        + [pltpu.VMEM((P,), jnp.float32)] * 4
        + [pltpu.VMEM((P, CW), jnp.int32)] * 4
        + [pltpu.VMEM((P, CW + 1), jnp.int32)]
        + [pltpu.VMEM((C, P), jnp.float32)]
    )
    return ([pltpu.VMEM((N_PER_W,), jnp.float32)] * 2
            + per_set * 2
            + [pltpu.SemaphoreType.DMA] * 8
            + [pltpu.SemaphoreType.DMA] * 2)


def _make_sc_kernel():
    mesh = plsc.VectorSubcoreMesh(core_axis_name="c", subcore_axis_name="s")

    @functools.partial(
        pl.kernel,
        mesh=mesh,
        compiler_params=pltpu.CompilerParams(
            use_tc_tiling_on_sc=False, needs_layout_passes=False),
        out_type=jax.ShapeDtypeStruct((C, N), jnp.float32),
        scratch_types=_scratch_types(),
    )
    def bilinear_sc(table_hbm, xs_hbm, ys_hbm, out_hbm, *s):
        it = iter(s)
        xall = next(it)
        yall = next(it)
        idx, wts, rows, ov, ovt = [], [], [], [], []
        for _ in range(2):
            idx.append([next(it) for _ in range(4)])
            wts.append([next(it) for _ in range(4)])
            rows.append([next(it) for _ in range(4)])
            ov.append(next(it))
            ovt.append(next(it))
        sg = [[next(it) for _ in range(4)] for _ in range(2)]
        so = [next(it) for _ in range(2)]

        wid = lax.axis_index("s") * NC + lax.axis_index("c")
        base = wid * N_PER_W

        def fire(k, b):
            for i in range(P // LANES):
                sl = pl.ds(i * LANES, LANES)
                cs = pl.ds(k * P + i * LANES, LANES)
                x = xall[cs]
                y = yall[cs]
                x0i = jnp.minimum(jnp.maximum(x.astype(jnp.int32), 0), H - 1)
                y0i = jnp.minimum(jnp.maximum(y.astype(jnp.int32), 0), W - 1)
                x1i = jnp.minimum(x0i + 1, H - 1)
                y1i = jnp.minimum(y0i + 1, W - 1)
                xc = jnp.minimum(jnp.maximum(x, 0.0), float(H - 1))
                yc = jnp.minimum(jnp.maximum(y, 0.0), float(W - 1))
                x0f = x0i.astype(jnp.float32)
                x1f = x1i.astype(jnp.float32)
                y0f = y0i.astype(jnp.float32)
                y1f = y1i.astype(jnp.float32)
                idx[b][0][sl] = x0i * W + y0i
                idx[b][1][sl] = x0i * W + y1i
                idx[b][2][sl] = x1i * W + y0i
                idx[b][3][sl] = x1i * W + y1i
                wts[b][0][sl] = (x1f - xc) * (y1f - yc)
                wts[b][1][sl] = (x1f - xc) * (yc - y0f)
                wts[b][2][sl] = (xc - x0f) * (y1f - yc)
                wts[b][3][sl] = (xc - x0f) * (yc - y0f)
            for c in range(4):
                pltpu.async_copy(table_hbm.at[idx[b][c]], rows[b][c], sg[b][c])

        def wait_gathers(b):
            for c in range(4):
                pltpu.make_async_copy(
                    table_hbm.at[idx[b][c]], rows[b][c], sg[b][c]).wait()

        def combine(b):
            ar, br, cr, dr = rows[b]

            def grp_body(g, c2):
                gs = pl.ds(g * LANES, LANES)
                wa16 = wts[b][0][gs]
                wb16 = wts[b][1][gs]
                wc16 = wts[b][2][gs]
                wd16 = wts[b][3][gs]
                row0 = g * LANES
                for p in range(LANES):
                    wav = jnp.full((LANES,), wa16[p], jnp.float32)
                    wbv = jnp.full((LANES,), wb16[p], jnp.float32)
                    wcv = jnp.full((LANES,), wc16[p], jnp.float32)
                    wdv = jnp.full((LANES,), wd16[p], jnp.float32)
                    wabf = plsc.pack(wav, wav,
                                     format=plsc.PackFormat.INTERLEAVED)
                    wbbf = plsc.pack(wbv, wbv,
                                     format=plsc.PackFormat.INTERLEAVED)
                    wcbf = plsc.pack(wcv, wcv,
                                     format=plsc.PackFormat.INTERLEAVED)
                    wdbf = plsc.pack(wdv, wdv,
                                     format=plsc.PackFormat.INTERLEAVED)
                    r = row0 + p
                    for j in range(CW // LANES):
                        sj = pl.ds(j * LANES, LANES)
                        va = plsc.bitcast(ar[r, sj], jnp.bfloat16)
                        vb = plsc.bitcast(br[r, sj], jnp.bfloat16)
                        vc = plsc.bitcast(cr[r, sj], jnp.bfloat16)
                        vd = plsc.bitcast(dr[r, sj], jnp.bfloat16)
                        acc = (wabf * va + wbbf * vb
                               + wcbf * vc + wdbf * vd)
                        ov[b][r, sj] = plsc.bitcast(acc, jnp.int32)
                return c2

            lax.fori_loop(0, P // LANES, grp_body, 0)

        def transpose_block(b):
            @plsc.parallel_loop(0, CW, unroll=4)
            def _t_body(cp):
                cvec = jnp.full((LANES,), cp, jnp.int32)
                for g in range(P // LANES):
                    gs = pl.ds(g * LANES, LANES)
                    pvec = g * LANES + lax.iota(jnp.int32, LANES)
                    wv = plsc.load_gather(ov[b], [pvec, cvec])
                    ev, od = plsc.unpack(
                        plsc.bitcast(wv, jnp.bfloat16),
                        format=plsc.PackFormat.INTERLEAVED)
                    ovt[b][2 * cp, gs] = ev
                    ovt[b][2 * cp + 1, gs] = od

        def fire_out(k, b):
            off = base + k * P
            pltpu.async_copy(ovt[b], out_hbm.at[:, pl.ds(off, P)], so[b])

        def wait_out(b):
            pltpu.make_async_copy(
                ovt[b], out_hbm.at[:, pl.ds(base, P)], so[b]).wait()

        pltpu.sync_copy(xs_hbm.at[pl.ds(base, N_PER_W)], xall)
        pltpu.sync_copy(ys_hbm.at[pl.ds(base, N_PER_W)], yall)
        fire(0, 0)

        def pair_body(kk, carry):
            for bph in range(2):
                k = 2 * kk + bph

                @pl.when(k + 1 < NCHUNK)
                def _fire_next():
                    fire(k + 1, 1 - bph)

                wait_gathers(bph)

                @pl.when(k >= 2)
                def _drain_out():
                    wait_out(bph)

                combine(bph)
                transpose_block(bph)
                fire_out(k, bph)
            return carry

        lax.fori_loop(0, NCHUNK // 2, pair_body, 0)
        wait_out(0)
        wait_out(1)

    return bilinear_sc


_bilinear_sc = _make_sc_kernel()


def kernel(images, coordinates):
    B, c, h, w = images.shape
    table_bf = images.reshape(c, h * w).T.astype(jnp.bfloat16)
    table_words = jax.lax.bitcast_convert_type(
        table_bf.reshape(h * w, c // 2, 2), jnp.int32)
    xs = coordinates[:, 0].reshape(-1)
    ys = coordinates[:, 1].reshape(-1)
    out_cn = _bilinear_sc(table_words, xs, ys)
    new_h, new_w = coordinates.shape[2], coordinates.shape[3]
    return out_cn.reshape(B, c, new_h, new_w)


# speedup vs baseline: 2.5003x; 1.3616x over previous
import functools

import jax
import jax.numpy as jnp
from jax import lax
from jax.experimental import pallas as pl
from jax.experimental.pallas import tpu as pltpu
from jax.experimental.pallas import tpu_sc as plsc

H = 384
W = 384
C = 192
CW = C // 2
N = H * W
NC = 2
NS = 16
NW = NC * NS
LANES = 16
N_PER_W = N // NW
P = 64
NCHUNK = N_PER_W // P


def _scratch_types():
    per_set = (
        [pltpu.VMEM((P,), jnp.int32)] * 4
        + [pltpu.VMEM((P,), jnp.float32)] * 4
        + [pltpu.VMEM((P, C), jnp.bfloat16)] * 4
        + [pltpu.VMEM((P, CW + 1), jnp.int32)]
        + [pltpu.VMEM((C, P), jnp.float32)]
    )
    return ([pltpu.VMEM((N_PER_W,), jnp.float32)] * 2
            + per_set * 2
            + [pltpu.SemaphoreType.DMA] * 8
            + [pltpu.SemaphoreType.DMA] * 2)


def _make_sc_kernel():
    mesh = plsc.VectorSubcoreMesh(core_axis_name="c", subcore_axis_name="s")

    @functools.partial(
        pl.kernel,
        mesh=mesh,
        compiler_params=pltpu.CompilerParams(
            use_tc_tiling_on_sc=False, needs_layout_passes=False),
        out_type=jax.ShapeDtypeStruct((C, N), jnp.float32),
        scratch_types=_scratch_types(),
    )
    def bilinear_sc(table_hbm, xs_hbm, ys_hbm, out_hbm, *s):
        it = iter(s)
        xall = next(it)
        yall = next(it)
        idx, wts, rows, ov, ovt = [], [], [], [], []
        for _ in range(2):
            idx.append([next(it) for _ in range(4)])
            wts.append([next(it) for _ in range(4)])
            rows.append([next(it) for _ in range(4)])
            ov.append(next(it))
            ovt.append(next(it))
        sg = [[next(it) for _ in range(4)] for _ in range(2)]
        so = [next(it) for _ in range(2)]

        wid = lax.axis_index("s") * NC + lax.axis_index("c")
        base = wid * N_PER_W

        def fire(k, b):
            for i in range(P // LANES):
                sl = pl.ds(i * LANES, LANES)
                cs = pl.ds(k * P + i * LANES, LANES)
                x = xall[cs]
                y = yall[cs]
                x0i = jnp.minimum(jnp.maximum(x.astype(jnp.int32), 0), H - 1)
                y0i = jnp.minimum(jnp.maximum(y.astype(jnp.int32), 0), W - 1)
                x1i = jnp.minimum(x0i + 1, H - 1)
                y1i = jnp.minimum(y0i + 1, W - 1)
                xc = jnp.minimum(jnp.maximum(x, 0.0), float(H - 1))
                yc = jnp.minimum(jnp.maximum(y, 0.0), float(W - 1))
                x0f = x0i.astype(jnp.float32)
                x1f = x1i.astype(jnp.float32)
                y0f = y0i.astype(jnp.float32)
                y1f = y1i.astype(jnp.float32)
                idx[b][0][sl] = x0i * W + y0i
                idx[b][1][sl] = x0i * W + y1i
                idx[b][2][sl] = x1i * W + y0i
                idx[b][3][sl] = x1i * W + y1i
                wts[b][0][sl] = (x1f - xc) * (y1f - yc)
                wts[b][1][sl] = (x1f - xc) * (yc - y0f)
                wts[b][2][sl] = (xc - x0f) * (y1f - yc)
                wts[b][3][sl] = (xc - x0f) * (yc - y0f)
            for c in range(4):
                pltpu.async_copy(table_hbm.at[idx[b][c]], rows[b][c], sg[b][c])

        def wait_gathers(b):
            for c in range(4):
                pltpu.make_async_copy(
                    table_hbm.at[idx[b][c]], rows[b][c], sg[b][c]).wait()

        def combine(b):
            ar, br, cr, dr = rows[b]

            def grp_body(g, c2):
                gs = pl.ds(g * LANES, LANES)
                wa16 = wts[b][0][gs]
                wb16 = wts[b][1][gs]
                wc16 = wts[b][2][gs]
                wd16 = wts[b][3][gs]
                row0 = g * LANES
                for p in range(LANES):
                    wav = jnp.full((LANES,), wa16[p], jnp.float32)
                    wbv = jnp.full((LANES,), wb16[p], jnp.float32)
                    wcv = jnp.full((LANES,), wc16[p], jnp.float32)
                    wdv = jnp.full((LANES,), wd16[p], jnp.float32)
                    wabf = plsc.pack(wav, wav,
                                     format=plsc.PackFormat.INTERLEAVED)
                    wbbf = plsc.pack(wbv, wbv,
                                     format=plsc.PackFormat.INTERLEAVED)
                    wcbf = plsc.pack(wcv, wcv,
                                     format=plsc.PackFormat.INTERLEAVED)
                    wdbf = plsc.pack(wdv, wdv,
                                     format=plsc.PackFormat.INTERLEAVED)
                    r = row0 + p
                    for j in range(CW // LANES):
                        sj = pl.ds(j * 2 * LANES, 2 * LANES)
                        sw = pl.ds(j * LANES, LANES)
                        va = ar[r, sj]
                        vb = br[r, sj]
                        vc = cr[r, sj]
                        vd = dr[r, sj]
                        acc = (wabf * va + wbbf * vb
                               + wcbf * vc + wdbf * vd)
                        ov[b][r, sw] = plsc.bitcast(acc, jnp.int32)
                return c2

            lax.fori_loop(0, P // LANES, grp_body, 0)

        def transpose_block(b):
            @plsc.parallel_loop(0, CW, unroll=4)
            def _t_body(cp):
                cvec = jnp.full((LANES,), cp, jnp.int32)
                for g in range(P // LANES):
                    gs = pl.ds(g * LANES, LANES)
                    pvec = g * LANES + lax.iota(jnp.int32, LANES)
                    wv = plsc.load_gather(ov[b], [pvec, cvec])
                    ev, od = plsc.unpack(
                        plsc.bitcast(wv, jnp.bfloat16),
                        format=plsc.PackFormat.INTERLEAVED)
                    ovt[b][2 * cp, gs] = ev
                    ovt[b][2 * cp + 1, gs] = od

        def fire_out(k, b):
            off = base + k * P
            pltpu.async_copy(ovt[b], out_hbm.at[:, pl.ds(off, P)], so[b])

        def wait_out(b):
            pltpu.make_async_copy(
                ovt[b], out_hbm.at[:, pl.ds(base, P)], so[b]).wait()

        pltpu.sync_copy(xs_hbm.at[pl.ds(base, N_PER_W)], xall)
        pltpu.sync_copy(ys_hbm.at[pl.ds(base, N_PER_W)], yall)
        fire(0, 0)

        def pair_body(kk, carry):
            for bph in range(2):
                k = 2 * kk + bph

                @pl.when(k + 1 < NCHUNK)
                def _fire_next():
                    fire(k + 1, 1 - bph)

                wait_gathers(bph)

                @pl.when(k >= 2)
                def _drain_out():
                    wait_out(bph)

                combine(bph)
                transpose_block(bph)
                fire_out(k, bph)
            return carry

        lax.fori_loop(0, NCHUNK // 2, pair_body, 0)
        wait_out(0)
        wait_out(1)

    return bilinear_sc


_bilinear_sc = _make_sc_kernel()


def kernel(images, coordinates):
    B, c, h, w = images.shape
    table_bf = images.reshape(c, h * w).T.astype(jnp.bfloat16)
    xs = coordinates[:, 0].reshape(-1)
    ys = coordinates[:, 1].reshape(-1)
    out_cn = _bilinear_sc(table_bf, xs, ys)
    new_h, new_w = coordinates.shape[2], coordinates.shape[3]
    return out_cn.reshape(B, c, new_h, new_w)
